# baseline (speedup 1.0000x reference)
import functools

import jax
import jax.numpy as jnp
from jax import lax
from jax.experimental import pallas as pl
from jax.experimental.pallas import tpu as pltpu
from jax.experimental.pallas import tpu_sc as plsc

NC, NS, L = 2, 16, 16
NW = NC * NS

BATCH = 4096
SEQ = 200
SEQP = 208
EMBED_DIM = 64
PADD = 128
PER_W = BATCH // NW
GROUPS = PER_W // 2


def _pos_encoding(seq_len, d):
    position = jnp.arange(0, seq_len, dtype=jnp.float32)[:, None]
    div_term = jnp.exp(jnp.arange(0, d, 2, dtype=jnp.float32) * -(jnp.log(10000.0) / d))
    enc = jnp.zeros((seq_len, d), dtype=jnp.float32)
    enc = enc.at[:, 0::2].set(jnp.sin(position * div_term))
    enc = enc.at[:, 1::2].set(jnp.cos(position * div_term[: d // 2]))
    return enc


def _body(weight_hbm, idx_hbm, enc_hbm, out_hbm, enc_v,
          idxb0, idxb1, gbuf0, gbuf1, sbuf0, sbuf1,
          i0, i1, g0, g1, s0, s1):
    idxbs = [idxb0, idxb1]
    gbufs = [gbuf0, gbuf1]
    sbufs = [sbuf0, sbuf1]
    isems = [i0, i1]
    gsems = [g0, g1]
    ssems = [s0, s1]
    wid = lax.axis_index("s") * NC + lax.axis_index("c")
    row0 = wid * PER_W

    pltpu.sync_copy(enc_hbm, enc_v)

    def start_idx(c, b):
        pltpu.async_copy(idx_hbm.at[row0 + c], idxbs[b], isems[b])

    def wait_idx(c, b):
        pltpu.make_async_copy(idx_hbm.at[row0 + c], idxbs[b], isems[b]).wait()

    def start_gather(b):
        for k in range(SEQP // L):
            iv = idxbs[b][pl.ds(k * L, L)]
            pltpu.async_copy(
                weight_hbm.at[plsc.Indices(iv)],
                gbufs[b].at[pl.ds(k * L, L), :],
                gsems[b],
            )

    def wait_gather(b):
        for k in range(SEQP // L):
            iv = idxbs[b][pl.ds(k * L, L)]
            pltpu.make_async_copy(
                weight_hbm.at[plsc.Indices(iv)],
                gbufs[b].at[pl.ds(k * L, L), :],
                gsems[b],
            ).wait()

    def start_store(c, b):
        pltpu.async_copy(sbufs[b], out_hbm.at[row0 + c], ssems[b])

    def wait_store(c, b):
        pltpu.make_async_copy(sbufs[b], out_hbm.at[row0 + c], ssems[b]).wait()

    start_idx(0, 0)
    wait_idx(0, 0)
    start_gather(0)
    start_idx(1, 1)

    def group_body(g, carry):
        for b in range(2):
            c = g * 2 + b
            ob = 1 - b
            wait_gather(b)
            if b == 0:
                wait_idx(c + 1, ob)
                start_gather(ob)

                @pl.when(g < GROUPS - 1)
                def _():
                    start_idx(c + 2, b)

            else:

                @pl.when(g < GROUPS - 1)
                def _():
                    wait_idx(c + 1, ob)
                    start_gather(ob)
                    start_idx(c + 2, b)

            @pl.when(g > 0)
            def _():
                wait_store(c - 2, b)

            gv = gbufs[b]
            sv = sbufs[b]

            @plsc.parallel_loop(0, SEQ // 2, unroll=4)
            def _(p):
                for j in range(EMBED_DIM // L):
                    sl = pl.ds(j * L, L)
                    sh = pl.ds(EMBED_DIM + j * L, L)
                    sv[2 * p, sl] = gv[2 * p, sl] + enc_v[p, sl]
                    sv[2 * p + 1, sl] = gv[2 * p + 1, sl] + enc_v[p, sh]

            start_store(c, b)
        return carry

    lax.fori_loop(0, GROUPS, group_body, 0)
    wait_store(PER_W - 2, 0)
    wait_store(PER_W - 1, 1)


@jax.jit
def _embed(text, weight, enc_pairs):
    wpad = jnp.pad(weight, ((0, 0), (0, PADD - EMBED_DIM)))
    idxp = jnp.pad(text.astype(jnp.int32), ((0, 0), (0, SEQP - SEQ)))
    mesh = plsc.VectorSubcoreMesh(
        core_axis_name="c", subcore_axis_name="s", num_cores=NC, num_subcores=NS
    )
    return pl.kernel(
        _body,
        out_type=jax.ShapeDtypeStruct((BATCH, SEQ, EMBED_DIM), jnp.float32),
        mesh=mesh,
        scratch_types=[
            pltpu.VMEM((SEQ // 2, 2 * EMBED_DIM), jnp.float32),
            pltpu.VMEM((SEQP,), jnp.int32),
            pltpu.VMEM((SEQP,), jnp.int32),
            pltpu.VMEM((SEQP, PADD), jnp.float32),
            pltpu.VMEM((SEQP, PADD), jnp.float32),
            pltpu.VMEM((SEQ, EMBED_DIM), jnp.float32),
            pltpu.VMEM((SEQ, EMBED_DIM), jnp.float32),
        ]
        + [pltpu.SemaphoreType.DMA for _ in range(6)],
    )(wpad, idxp, enc_pairs)


def kernel(text, weight):
    enc = _pos_encoding(SEQ, EMBED_DIM)
    enc_pairs = enc.reshape(SEQ // 2, 2 * EMBED_DIM)
    return _embed(text, weight, enc_pairs)


# speedup vs baseline: 2.6849x; 2.6849x over previous
import functools

import jax
import jax.numpy as jnp
from jax import lax
from jax.experimental import pallas as pl
from jax.experimental.pallas import tpu as pltpu
from jax.experimental.pallas import tpu_sc as plsc

NC, NS, L = 2, 16, 16
NW = NC * NS

BATCH = 4096
SEQ = 200
EMBED_DIM = 64
TOTAL = BATCH * SEQ
PER_W = TOTAL // NW
PADD = 128
BLK = 128
NBLK = PER_W // BLK
GROUPS = NBLK // 2
ENC_ROWS = SEQ + BLK - 8


def _pos_encoding(seq_len, d):
    position = jnp.arange(0, seq_len, dtype=jnp.float32)[:, None]
    div_term = jnp.exp(jnp.arange(0, d, 2, dtype=jnp.float32) * -(jnp.log(10000.0) / d))
    enc = jnp.zeros((seq_len, d), dtype=jnp.float32)
    enc = enc.at[:, 0::2].set(jnp.sin(position * div_term))
    enc = enc.at[:, 1::2].set(jnp.cos(position * div_term[: d // 2]))
    return enc


def _body(weight_hbm, idx_hbm, enc_hbm, out_hbm, idx_all, enc_v,
          gbuf0, gbuf1, sbuf0, sbuf1, g0, g1, s0, s1):
    gbufs = [gbuf0, gbuf1]
    sbufs = [sbuf0, sbuf1]
    gsems = [g0, g1]
    ssems = [s0, s1]
    wid = lax.axis_index("s") * NC + lax.axis_index("c")
    base = wid * PER_W

    pltpu.sync_copy(idx_hbm.at[pl.ds(wid * NBLK, NBLK), :], idx_all)
    pltpu.sync_copy(enc_hbm, enc_v)

    def start_gather(c, b):
        for k in range(BLK // L):
            iv = idx_all[c, pl.ds(k * L, L)]
            pltpu.async_copy(
                weight_hbm.at[plsc.Indices(iv)],
                gbufs[b].at[pl.ds(k * L, L), :],
                gsems[b],
            )

    def wait_gather(c, b):
        for k in range(BLK // L):
            iv = idx_all[c, pl.ds(k * L, L)]
            pltpu.make_async_copy(
                weight_hbm.at[plsc.Indices(iv)],
                gbufs[b].at[pl.ds(k * L, L), :],
                gsems[b],
            ).wait()

    def start_store(c, b):
        pltpu.async_copy(
            sbufs[b], out_hbm.at[pl.ds(base + c * BLK, BLK)], ssems[b]
        )

    def wait_store(c, b):
        pltpu.make_async_copy(
            sbufs[b], out_hbm.at[pl.ds(base + c * BLK, BLK)], ssems[b]
        ).wait()

    start_gather(0, 0)

    def group_body(g, carry):
        for b in range(2):
            c = g * 2 + b
            ob = 1 - b
            wait_gather(c, b)
            if b == 0:
                start_gather(c + 1, ob)
            else:

                @pl.when(g < GROUPS - 1)
                def _():
                    start_gather(c + 1, ob)

            @pl.when(g > 0)
            def _():
                wait_store(c - 2, b)

            off2 = ((c * BLK) % SEQ) // 2
            gv = gbufs[b]
            sv = sbufs[b]

            @plsc.parallel_loop(0, BLK // 2, unroll=4)
            def _(p):
                e = off2 + p
                for j in range(EMBED_DIM // L):
                    sl = pl.ds(j * L, L)
                    sh = pl.ds(EMBED_DIM + j * L, L)
                    sv[2 * p, sl] = gv[2 * p, sl] + enc_v[e, sl]
                    sv[2 * p + 1, sl] = gv[2 * p + 1, sl] + enc_v[e, sh]

            start_store(c, b)
        return carry

    lax.fori_loop(0, GROUPS, group_body, 0)
    wait_store(NBLK - 2, 0)
    wait_store(NBLK - 1, 1)


@jax.jit
def _embed(text, weight, enc_pairs):
    wpad = jnp.pad(weight, ((0, 0), (0, PADD - EMBED_DIM)))
    idx2d = text.reshape(NBLK * NW, BLK).astype(jnp.int32)
    mesh = plsc.VectorSubcoreMesh(
        core_axis_name="c", subcore_axis_name="s", num_cores=NC, num_subcores=NS
    )
    out = pl.kernel(
        _body,
        out_type=jax.ShapeDtypeStruct((TOTAL, EMBED_DIM), jnp.float32),
        mesh=mesh,
        scratch_types=[
            pltpu.VMEM((NBLK, BLK), jnp.int32),
            pltpu.VMEM((ENC_ROWS // 2, 2 * EMBED_DIM), jnp.float32),
            pltpu.VMEM((BLK, PADD), jnp.float32),
            pltpu.VMEM((BLK, PADD), jnp.float32),
            pltpu.VMEM((BLK, EMBED_DIM), jnp.float32),
            pltpu.VMEM((BLK, EMBED_DIM), jnp.float32),
        ]
        + [pltpu.SemaphoreType.DMA for _ in range(4)],
    )(wpad, idx2d, enc_pairs)
    return out.reshape(BATCH, SEQ, EMBED_DIM)


def kernel(text, weight):
    enc = _pos_encoding(SEQ, EMBED_DIM)
    enc_ext = jnp.concatenate([enc, enc[: ENC_ROWS - SEQ]], axis=0)
    enc_pairs = enc_ext.reshape(ENC_ROWS // 2, 2 * EMBED_DIM)
    return _embed(text, weight, enc_pairs)


# speedup vs baseline: 2.8159x; 1.0488x over previous
import functools

import jax
import jax.numpy as jnp
from jax import lax
from jax.experimental import pallas as pl
from jax.experimental.pallas import tpu as pltpu
from jax.experimental.pallas import tpu_sc as plsc

NC, NS, L = 2, 16, 16
NW = NC * NS

BATCH = 4096
SEQ = 200
EMBED_DIM = 64
TOTAL = BATCH * SEQ
PER_W = TOTAL // NW
PADD = 128
BLK = 128
NBLK = PER_W // BLK
GROUPS = NBLK // 4
ENC_ROWS = SEQ + BLK - 8


def _pos_encoding(seq_len, d):
    position = jnp.arange(0, seq_len, dtype=jnp.float32)[:, None]
    div_term = jnp.exp(jnp.arange(0, d, 2, dtype=jnp.float32) * -(jnp.log(10000.0) / d))
    enc = jnp.zeros((seq_len, d), dtype=jnp.float32)
    enc = enc.at[:, 0::2].set(jnp.sin(position * div_term))
    enc = enc.at[:, 1::2].set(jnp.cos(position * div_term[: d // 2]))
    return enc


def _body(weight_hbm, idx_hbm, enc_hbm, out_hbm, enc_v,
          ib0, ib1, ib2, ib3, gbuf0, gbuf1, gbuf2, gbuf3, sbuf0, sbuf1,
          i0, i1, i2, i3, g0, g1, g2, g3, s0, s1):
    idxbs = [ib0, ib1, ib2, ib3]
    gbufs = [gbuf0, gbuf1, gbuf2, gbuf3]
    sbufs = [sbuf0, sbuf1]
    isems = [i0, i1, i2, i3]
    gsems = [g0, g1, g2, g3]
    ssems = [s0, s1]
    wid = lax.axis_index("s") * NC + lax.axis_index("c")
    base = wid * PER_W

    pltpu.sync_copy(enc_hbm, enc_v)

    def start_idx(c, b):
        pltpu.async_copy(idx_hbm.at[wid * NBLK + c], idxbs[b], isems[b])

    def wait_idx(c, b):
        pltpu.make_async_copy(
            idx_hbm.at[wid * NBLK + c], idxbs[b], isems[b]
        ).wait()

    def start_gather(b):
        for k in range(BLK // L):
            iv = idxbs[b][pl.ds(k * L, L)]
            pltpu.async_copy(
                weight_hbm.at[plsc.Indices(iv)],
                gbufs[b].at[pl.ds(k * L, L), :],
                gsems[b],
            )

    def wait_gather(b):
        for k in range(BLK // L):
            iv = idxbs[b][pl.ds(k * L, L)]
            pltpu.make_async_copy(
                weight_hbm.at[plsc.Indices(iv)],
                gbufs[b].at[pl.ds(k * L, L), :],
                gsems[b],
            ).wait()

    def start_store(c, b):
        pltpu.async_copy(
            sbufs[b], out_hbm.at[pl.ds(base + c * BLK, BLK)], ssems[b]
        )

    def wait_store(c, b):
        pltpu.make_async_copy(
            sbufs[b], out_hbm.at[pl.ds(base + c * BLK, BLK)], ssems[b]
        ).wait()

    for b in range(4):
        start_idx(b, b)
    wait_idx(0, 0)
    start_gather(0)
    wait_idx(1, 1)
    start_gather(1)

    def group_body(g, carry):
        for b in range(4):
            c = g * 4 + b
            sb = b % 2
            nb = (b + 2) % 4
            wait_gather(b)
            if b < 2:
                wait_idx(c + 2, nb)
                start_gather(nb)
            else:

                @pl.when(g < GROUPS - 1)
                def _():
                    wait_idx(c + 2, nb)
                    start_gather(nb)

            @pl.when(g < GROUPS - 1)
            def _():
                start_idx(c + 4, b)

            if b < 2:

                @pl.when(g > 0)
                def _():
                    wait_store(c - 2, sb)

            else:
                wait_store(c - 2, sb)

            off2 = ((c * BLK) % SEQ) // 2
            gv = gbufs[b]
            sv = sbufs[sb]

            @plsc.parallel_loop(0, BLK // 2, unroll=4)
            def _(p):
                e = off2 + p
                for j in range(EMBED_DIM // L):
                    sl = pl.ds(j * L, L)
                    sh = pl.ds(EMBED_DIM + j * L, L)
                    sv[2 * p, sl] = gv[2 * p, sl] + enc_v[e, sl]
                    sv[2 * p + 1, sl] = gv[2 * p + 1, sl] + enc_v[e, sh]

            start_store(c, sb)
        return carry

    lax.fori_loop(0, GROUPS, group_body, 0)
    wait_store(NBLK - 2, 0)
    wait_store(NBLK - 1, 1)


@jax.jit
def _embed(text, weight, enc_pairs):
    wpad = jnp.pad(weight, ((0, 0), (0, PADD - EMBED_DIM)))
    idx2d = text.reshape(NBLK * NW, BLK).astype(jnp.int32)
    mesh = plsc.VectorSubcoreMesh(
        core_axis_name="c", subcore_axis_name="s", num_cores=NC, num_subcores=NS
    )
    out = pl.kernel(
        _body,
        out_type=jax.ShapeDtypeStruct((TOTAL, EMBED_DIM), jnp.float32),
        mesh=mesh,
        scratch_types=[
            pltpu.VMEM((ENC_ROWS // 2, 2 * EMBED_DIM), jnp.float32),
        ]
        + [pltpu.VMEM((BLK,), jnp.int32) for _ in range(4)]
        + [pltpu.VMEM((BLK, PADD), jnp.float32) for _ in range(4)]
        + [pltpu.VMEM((BLK, EMBED_DIM), jnp.float32) for _ in range(2)]
        + [pltpu.SemaphoreType.DMA for _ in range(10)],
    )(wpad, idx2d, enc_pairs)
    return out.reshape(BATCH, SEQ, EMBED_DIM)


def kernel(text, weight):
    enc = _pos_encoding(SEQ, EMBED_DIM)
    enc_ext = jnp.concatenate([enc, enc[: ENC_ROWS - SEQ]], axis=0)
    enc_pairs = enc_ext.reshape(ENC_ROWS // 2, 2 * EMBED_DIM)
    return _embed(text, weight, enc_pairs)
